# Initial kernel scaffold; baseline (speedup 1.0000x reference)
import functools
import jax
import jax.numpy as jnp
from jax import lax
from jax.experimental import pallas as pl
from jax.experimental.pallas import tpu as pltpu
from jax.experimental.pallas import tpu_sc as plsc

_N = 10000
_E = 320000
_D = 128
_G = 64

_NC = 2
_NS = 16
_NW = _NC * _NS
_EW = _E // _NW
_K = 80
_NCHUNK = _EW // _K
_RPT = _N // _NS
_ZR = 125

_mesh = plsc.VectorSubcoreMesh(core_axis_name="c", subcore_axis_name="s")


def _deg_body(dst_hbm, w_hbm, degp_hbm, dst_v, w_v, deg_v):
    c = lax.axis_index("c")
    s = lax.axis_index("s")
    wid = s * _NC + c
    base = wid * _EW
    pltpu.sync_copy(dst_hbm.at[pl.ds(base, _EW)], dst_v)
    pltpu.sync_copy(w_hbm.at[pl.ds(base, _EW)], w_v)

    zeros = jnp.zeros((16,), jnp.float32)

    def _zero(i, carry):
        deg_v[pl.ds(i * 16, 16)] = zeros
        return carry

    lax.fori_loop(0, _N // 16, _zero, 0)

    def _edge(i, carry):
        d = dst_v[i]
        deg_v[d] = deg_v[d] + w_v[i]
        return carry

    lax.fori_loop(0, _EW, _edge, 0)
    pltpu.sync_copy(deg_v, degp_hbm.at[wid])


_deg_kernel = functools.partial(
    pl.kernel,
    out_type=jax.ShapeDtypeStruct((_NW, _N), jnp.float32),
    mesh=_mesh,
    scratch_types=[
        pltpu.VMEM((_EW,), jnp.int32),
        pltpu.VMEM((_EW,), jnp.float32),
        pltpu.VMEM((_N,), jnp.float32),
    ],
)(_deg_body)


def _agg_body(y_hbm, src_hbm, dst_hbm, w_hbm, out_hbm,
              src_v, dst_v, w_v, rows_v, zbuf_v, sem, acc):
    c = lax.axis_index("c")
    s = lax.axis_index("s")
    wid = s * _NC + c

    pltpu.sync_copy(src_hbm.at[wid], src_v)
    pltpu.sync_copy(dst_hbm.at[wid], dst_v)
    pltpu.sync_copy(w_hbm.at[pl.ds(wid * _EW, _EW)], w_v)

    zeros = jnp.zeros((16,), jnp.float32)

    def _zero(i, carry):
        for dd in range(_D // 16):
            zbuf_v[i, pl.ds(16 * dd, 16)] = zeros
        return carry

    lax.fori_loop(0, _ZR, _zero, 0)
    for r in range(_RPT // _ZR):
        pltpu.sync_copy(zbuf_v, acc.at[pl.ds(s * _RPT + r * _ZR, _ZR)])
    plsc.subcore_barrier()

    def _chunk(j, carry):
        pltpu.async_copy(y_hbm.at[src_v.at[j]], rows_v, sem).wait()

        def _scale(e, c2):
            w = w_v[j * _K + e]
            wv = jnp.full((16,), w, jnp.float32)
            for dd in range(_D // 16):
                sl = pl.ds(16 * dd, 16)
                rows_v[e, sl] = rows_v[e, sl] * wv
            return c2

        lax.fori_loop(0, _K, _scale, 0)

        pltpu.sync_copy(rows_v, acc.at[dst_v.at[j]], add=True)
        return carry

    lax.fori_loop(0, _NCHUNK, _chunk, 0)
    plsc.subcore_barrier()

    pltpu.sync_copy(acc.at[pl.ds(s * _RPT, _RPT)],
                    out_hbm.at[c, pl.ds(s * _RPT, _RPT)])


_agg_kernel = functools.partial(
    pl.kernel,
    out_type=jax.ShapeDtypeStruct((_NC, _N, _D), jnp.float32),
    mesh=_mesh,
    scratch_types=[
        pltpu.VMEM((_NCHUNK, _K), jnp.int32),
        pltpu.VMEM((_NCHUNK, _K), jnp.int32),
        pltpu.VMEM((_EW,), jnp.float32),
        pltpu.VMEM((_K, _D), jnp.float32),
        pltpu.VMEM((_ZR, _D), jnp.float32),
        pltpu.SemaphoreType.DMA,
        pltpu.VMEM_SHARED((_N, _D), jnp.float32),
    ],
)(_agg_body)


def _pre_body(degp_ref, x_ref, wfc_ref, w0_ref, dis_ref, y0_ref):
    ones = jnp.ones((_NW, 1), jnp.float32)
    deg = lax.dot_general(degp_ref[...], ones, (((0,), (0,)), ((), ())),
                          preferred_element_type=jnp.float32) + 1.0
    dis = jnp.where(deg > 0, lax.rsqrt(jnp.maximum(deg, 1e-12)), 0.0)
    h0 = jnp.dot(x_ref[...], wfc_ref[...].T, preferred_element_type=jnp.float32)
    y0 = dis * jnp.dot(h0, w0_ref[...].T, preferred_element_type=jnp.float32)
    dis_ref[...] = dis
    y0_ref[...] = y0


_pre_kernel = pl.pallas_call(
    _pre_body,
    out_shape=(
        jax.ShapeDtypeStruct((_N, 1), jnp.float32),
        jax.ShapeDtypeStruct((_N, _D), jnp.float32),
    ),
)


def _post_body(p_ref, y_ref, dis_ref, b_ref, g_ref, be_ref, wn_ref,
               h_ref, yn_ref):
    agg = p_ref[0] + p_ref[1] + y_ref[...]
    pre = dis_ref[...] * agg + b_ref[...]
    r = jnp.maximum(pre, 0.0)
    mu = jnp.mean(r, axis=0, keepdims=True)
    d = r - mu
    var = jnp.mean(d * d, axis=0, keepdims=True)
    h = g_ref[...] * d * lax.rsqrt(var + 1e-5) + be_ref[...]
    h_ref[...] = h
    yn_ref[...] = dis_ref[...] * jnp.dot(h, wn_ref[...].T,
                                         preferred_element_type=jnp.float32)


_post_kernel = pl.pallas_call(
    _post_body,
    out_shape=(
        jax.ShapeDtypeStruct((_N, _D), jnp.float32),
        jax.ShapeDtypeStruct((_N, _D), jnp.float32),
    ),
)


def _final_body(p_ref, y_ref, dis_ref, b_ref, g_ref, be_ref,
                batch_ref, h0_ref, h1_ref, h2_ref, xout_ref):
    agg = p_ref[0] + p_ref[1] + y_ref[...]
    pre = dis_ref[...] * agg + b_ref[...]
    r = jnp.maximum(pre, 0.0)
    mu = jnp.mean(r, axis=0, keepdims=True)
    d = r - mu
    var = jnp.mean(d * d, axis=0, keepdims=True)
    h2 = g_ref[...] * d * lax.rsqrt(var + 1e-5) + be_ref[...]
    h2_ref[...] = h2

    hsum = h0_ref[...] + h1_ref[...] + h2
    ids = lax.broadcasted_iota(jnp.int32, (_G, _N), 0)
    sel = (batch_ref[...] == ids).astype(jnp.float32)
    xout_ref[...] = jnp.dot(sel, hsum, preferred_element_type=jnp.float32)


_final_kernel = pl.pallas_call(
    _final_body,
    out_shape=(
        jax.ShapeDtypeStruct((_N, _D), jnp.float32),
        jax.ShapeDtypeStruct((_G, _D), jnp.float32),
    ),
)


@jax.jit
def kernel(x, edge_index, edge_weight, batch, W_fc,
           W0, b0, g0, be0, W1, b1, g1, be1, W2, b2, g2, be2):
    src3 = edge_index[0].reshape(_NW, _NCHUNK, _K)
    dst3 = edge_index[1].reshape(_NW, _NCHUNK, _K)

    degp = _deg_kernel(edge_index[1], edge_weight)
    dis, y0 = _pre_kernel(degp, x, W_fc, W0)

    p0 = _agg_kernel(y0, src3, dst3, edge_weight)
    h0, y1 = _post_kernel(p0, y0, dis, b0.reshape(1, _D), g0.reshape(1, _D),
                          be0.reshape(1, _D), W1)

    p1 = _agg_kernel(y1, src3, dst3, edge_weight)
    h1, y2 = _post_kernel(p1, y1, dis, b1.reshape(1, _D), g1.reshape(1, _D),
                          be1.reshape(1, _D), W2)

    p2 = _agg_kernel(y2, src3, dst3, edge_weight)
    h2, xout = _final_kernel(p2, y2, dis, b2.reshape(1, _D),
                             g2.reshape(1, _D), be2.reshape(1, _D),
                             batch.reshape(1, _N), h0, h1)

    return (xout, jnp.concatenate([h0, h1, h2], axis=1))


# speedup vs baseline: 12.9182x; 12.9182x over previous
import functools
import jax
import jax.numpy as jnp
from jax import lax
from jax.experimental import pallas as pl
from jax.experimental.pallas import tpu as pltpu
from jax.experimental.pallas import tpu_sc as plsc

_N = 10000
_E = 320000
_D = 128
_G = 64

_NC = 2
_NS = 16
_NW = _NC * _NS
_EW = _E // _NW
_K = 80
_NCHUNK = _EW // _K
_RPT = _N // _NS
_ZR = 125

_mesh = plsc.VectorSubcoreMesh(core_axis_name="c", subcore_axis_name="s")
_sc_params = pltpu.CompilerParams(use_tc_tiling_on_sc=False)


def _deg_body(dst_hbm, w_hbm, degp_hbm, dst_v, w_v, deg_v):
    c = lax.axis_index("c")
    s = lax.axis_index("s")
    wid = s * _NC + c
    base = wid * _EW
    pltpu.sync_copy(dst_hbm.at[pl.ds(base, _EW)], dst_v)
    pltpu.sync_copy(w_hbm.at[pl.ds(base, _EW)], w_v)

    zeros = jnp.zeros((16,), jnp.float32)
    lane = lax.broadcasted_iota(jnp.int32, (16,), 0)
    onehot0 = jnp.where(lane == 0, 1.0, 0.0).astype(jnp.float32)

    def _zero(i, carry):
        deg_v[pl.ds(i * 16, 16)] = zeros
        return carry

    lax.fori_loop(0, (_N + 16) // 16, _zero, 0)

    def _edge16(i, carry):
        d16 = dst_v[pl.ds(i * 16, 16)]
        w16 = w_v[pl.ds(i * 16, 16)]
        for l in range(16):
            d = d16[l]
            vec = deg_v[pl.ds(d, 16)]
            deg_v[pl.ds(d, 16)] = vec + onehot0 * w16[l]
        return carry

    lax.fori_loop(0, _EW // 16, _edge16, 0)
    pltpu.sync_copy(deg_v.at[pl.ds(0, _N)], degp_hbm.at[wid])


_deg_kernel = functools.partial(
    pl.kernel,
    out_type=jax.ShapeDtypeStruct((_NW, _N), jnp.float32),
    mesh=_mesh,
    compiler_params=_sc_params,
    scratch_types=[
        pltpu.VMEM((_EW,), jnp.int32),
        pltpu.VMEM((_EW,), jnp.float32),
        pltpu.VMEM((_N + 16,), jnp.float32),
    ],
)(_deg_body)


def _agg_body(y_hbm, src_hbm, dst_hbm, w_hbm, out_hbm,
              src_v, dst_v, w_v, rows_v, sem, acc):
    c = lax.axis_index("c")
    s = lax.axis_index("s")
    wid = s * _NC + c

    pltpu.sync_copy(src_hbm.at[wid], src_v)
    pltpu.sync_copy(dst_hbm.at[wid], dst_v)
    pltpu.sync_copy(w_hbm.at[pl.ds(wid * _EW, _EW)], w_v)

    zeros = jnp.zeros((16,), jnp.float32)

    def _zero(i, carry):
        for dd in range(_D // 16):
            rows_v[i, pl.ds(16 * dd, 16)] = zeros
        return carry

    lax.fori_loop(0, _K, _zero, 0)
    for r in range(_RPT // _K):
        pltpu.sync_copy(rows_v, acc.at[pl.ds(s * _RPT + r * _K, _K)])
    rem = _RPT - (_RPT // _K) * _K
    if rem:
        pltpu.sync_copy(rows_v.at[pl.ds(0, rem)],
                        acc.at[pl.ds(s * _RPT + (_RPT // _K) * _K, rem)])
    plsc.subcore_barrier()

    def _chunk(j, carry):
        pltpu.async_copy(y_hbm.at[src_v.at[j]], rows_v, sem).wait()

        def _scale(i, c2):
            w16 = w_v[pl.ds(j * _K + i * 16, 16)]
            for l in range(16):
                e = i * 16 + l
                w = w16[l]
                for dd in range(_D // 16):
                    sl = pl.ds(16 * dd, 16)
                    rows_v[e, sl] = rows_v[e, sl] * w
            return c2

        lax.fori_loop(0, _K // 16, _scale, 0)

        pltpu.sync_copy(rows_v, acc.at[dst_v.at[j]], add=True)
        return carry

    lax.fori_loop(0, _NCHUNK, _chunk, 0)
    plsc.subcore_barrier()

    pltpu.sync_copy(acc.at[pl.ds(s * _RPT, _RPT)],
                    out_hbm.at[c, pl.ds(s * _RPT, _RPT)])


_agg_kernel = functools.partial(
    pl.kernel,
    out_type=jax.ShapeDtypeStruct((_NC, _N, _D), jnp.float32),
    mesh=_mesh,
    compiler_params=_sc_params,
    scratch_types=[
        pltpu.VMEM((_NCHUNK, _K), jnp.int32),
        pltpu.VMEM((_NCHUNK, _K), jnp.int32),
        pltpu.VMEM((_EW,), jnp.float32),
        pltpu.VMEM((_K, _D), jnp.float32),
        pltpu.SemaphoreType.DMA,
        pltpu.VMEM_SHARED((_N, _D), jnp.float32),
    ],
)(_agg_body)


def _pre_body(degp_ref, x_ref, wfc_ref, w0_ref, dis_ref, y0_ref):
    ones = jnp.ones((_NW, 1), jnp.float32)
    deg = lax.dot_general(degp_ref[...], ones, (((0,), (0,)), ((), ())),
                          preferred_element_type=jnp.float32) + 1.0
    dis = jnp.where(deg > 0, lax.rsqrt(jnp.maximum(deg, 1e-12)), 0.0)
    h0 = jnp.dot(x_ref[...], wfc_ref[...].T, preferred_element_type=jnp.float32)
    y0 = dis * jnp.dot(h0, w0_ref[...].T, preferred_element_type=jnp.float32)
    dis_ref[...] = dis
    y0_ref[...] = y0


_pre_kernel = pl.pallas_call(
    _pre_body,
    out_shape=(
        jax.ShapeDtypeStruct((_N, 1), jnp.float32),
        jax.ShapeDtypeStruct((_N, _D), jnp.float32),
    ),
)


def _post_body(p_ref, y_ref, dis_ref, b_ref, g_ref, be_ref, wn_ref,
               h_ref, yn_ref):
    agg = p_ref[0] + p_ref[1] + y_ref[...]
    pre = dis_ref[...] * agg + b_ref[...]
    r = jnp.maximum(pre, 0.0)
    mu = jnp.mean(r, axis=0, keepdims=True)
    d = r - mu
    var = jnp.mean(d * d, axis=0, keepdims=True)
    h = g_ref[...] * d * lax.rsqrt(var + 1e-5) + be_ref[...]
    h_ref[...] = h
    yn_ref[...] = dis_ref[...] * jnp.dot(h, wn_ref[...].T,
                                         preferred_element_type=jnp.float32)


_post_kernel = pl.pallas_call(
    _post_body,
    out_shape=(
        jax.ShapeDtypeStruct((_N, _D), jnp.float32),
        jax.ShapeDtypeStruct((_N, _D), jnp.float32),
    ),
)


def _final_body(p_ref, y_ref, dis_ref, b_ref, g_ref, be_ref,
                batch_ref, h0_ref, h1_ref, h2_ref, xout_ref):
    agg = p_ref[0] + p_ref[1] + y_ref[...]
    pre = dis_ref[...] * agg + b_ref[...]
    r = jnp.maximum(pre, 0.0)
    mu = jnp.mean(r, axis=0, keepdims=True)
    d = r - mu
    var = jnp.mean(d * d, axis=0, keepdims=True)
    h2 = g_ref[...] * d * lax.rsqrt(var + 1e-5) + be_ref[...]
    h2_ref[...] = h2

    hsum = h0_ref[...] + h1_ref[...] + h2
    ids = lax.broadcasted_iota(jnp.int32, (_G, _N), 0)
    sel = (batch_ref[...] == ids).astype(jnp.float32)
    xout_ref[...] = jnp.dot(sel, hsum, preferred_element_type=jnp.float32)


_final_kernel = pl.pallas_call(
    _final_body,
    out_shape=(
        jax.ShapeDtypeStruct((_N, _D), jnp.float32),
        jax.ShapeDtypeStruct((_G, _D), jnp.float32),
    ),
)


@jax.jit
def kernel(x, edge_index, edge_weight, batch, W_fc,
           W0, b0, g0, be0, W1, b1, g1, be1, W2, b2, g2, be2):
    src3 = edge_index[0].reshape(_NW, _NCHUNK, _K)
    dst3 = edge_index[1].reshape(_NW, _NCHUNK, _K)

    degp = _deg_kernel(edge_index[1], edge_weight)
    dis, y0 = _pre_kernel(degp, x, W_fc, W0)

    p0 = _agg_kernel(y0, src3, dst3, edge_weight)
    h0, y1 = _post_kernel(p0, y0, dis, b0.reshape(1, _D), g0.reshape(1, _D),
                          be0.reshape(1, _D), W1)

    p1 = _agg_kernel(y1, src3, dst3, edge_weight)
    h1, y2 = _post_kernel(p1, y1, dis, b1.reshape(1, _D), g1.reshape(1, _D),
                          be1.reshape(1, _D), W2)

    p2 = _agg_kernel(y2, src3, dst3, edge_weight)
    h2, xout = _final_kernel(p2, y2, dis, b2.reshape(1, _D),
                             g2.reshape(1, _D), be2.reshape(1, _D),
                             batch.reshape(1, _N), h0, h1)

    return (xout, jnp.concatenate([h0, h1, h2], axis=1))


# speedup vs baseline: 20.5028x; 1.5871x over previous
import functools
import jax
import jax.numpy as jnp
from jax import lax
from jax.experimental import pallas as pl
from jax.experimental.pallas import tpu as pltpu
from jax.experimental.pallas import tpu_sc as plsc

_N = 10000
_E = 320000
_D = 128
_G = 64

_NC = 2
_NS = 16
_NW = _NC * _NS
_EW = _E // _NW
_K = 80
_NCHUNK = _EW // _K
_RPT = _N // _NS
_ZR = 125

_mesh = plsc.VectorSubcoreMesh(core_axis_name="c", subcore_axis_name="s")
_sc_params = pltpu.CompilerParams(use_tc_tiling_on_sc=False)


def _deg_body(dst_hbm, w_hbm, degp_hbm, dst_v, w_v, deg_v):
    c = lax.axis_index("c")
    s = lax.axis_index("s")
    wid = s * _NC + c
    base = wid * _EW
    pltpu.sync_copy(dst_hbm.at[pl.ds(base, _EW)], dst_v)
    pltpu.sync_copy(w_hbm.at[pl.ds(base, _EW)], w_v)

    zeros = jnp.zeros((16,), jnp.float32)
    lane = lax.broadcasted_iota(jnp.int32, (16,), 0)
    onehot0 = jnp.where(lane == 0, 1.0, 0.0).astype(jnp.float32)

    def _zero(i, carry):
        deg_v[pl.ds(i * 16, 16)] = zeros
        return carry

    lax.fori_loop(0, (_N + 16) // 16, _zero, 0)

    def _edge16(i, carry):
        d16 = dst_v[pl.ds(i * 16, 16)]
        w16 = w_v[pl.ds(i * 16, 16)]
        for l in range(16):
            d = d16[l]
            vec = deg_v[pl.ds(d, 16)]
            deg_v[pl.ds(d, 16)] = vec + onehot0 * w16[l]
        return carry

    lax.fori_loop(0, _EW // 16, _edge16, 0)
    pltpu.sync_copy(deg_v.at[pl.ds(0, _N)], degp_hbm.at[wid])


_deg_kernel = functools.partial(
    pl.kernel,
    out_type=jax.ShapeDtypeStruct((_NW, _N), jnp.float32),
    mesh=_mesh,
    compiler_params=_sc_params,
    scratch_types=[
        pltpu.VMEM((_EW,), jnp.int32),
        pltpu.VMEM((_EW,), jnp.float32),
        pltpu.VMEM((_N + 16,), jnp.float32),
    ],
)(_deg_body)


def _agg_body(y_hbm, src_hbm, dst_hbm, w_hbm, out_hbm,
              src_v, dst_v, w_v, rows_a, rows_b, sem_a, sem_b, acc):
    c = lax.axis_index("c")
    s = lax.axis_index("s")
    wid = s * _NC + c

    pltpu.sync_copy(src_hbm.at[wid], src_v)
    pltpu.sync_copy(dst_hbm.at[wid], dst_v)
    pltpu.sync_copy(w_hbm.at[pl.ds(wid * _EW, _EW)], w_v)

    zeros = jnp.zeros((16,), jnp.float32)

    def _zero(i, carry):
        for dd in range(_D // 16):
            rows_a[i, pl.ds(16 * dd, 16)] = zeros
        return carry

    lax.fori_loop(0, _K, _zero, 0)
    for r in range(_RPT // _K):
        pltpu.sync_copy(rows_a, acc.at[pl.ds(s * _RPT + r * _K, _K)])
    rem = _RPT - (_RPT // _K) * _K
    if rem:
        pltpu.sync_copy(rows_a.at[pl.ds(0, rem)],
                        acc.at[pl.ds(s * _RPT + (_RPT // _K) * _K, rem)])
    plsc.subcore_barrier()

    def _gather(j, buf, sem):
        return pltpu.make_async_copy(y_hbm.at[src_v.at[j]], buf, sem)

    def _process(j, buf, sem):
        _gather(j, buf, sem).wait()

        def _scale(i, c2):
            w16 = w_v[pl.ds(j * _K + i * 16, 16)]
            for l in range(16):
                e = i * 16 + l
                w = w16[l]
                for dd in range(_D // 16):
                    sl = pl.ds(16 * dd, 16)
                    buf[e, sl] = buf[e, sl] * w
            return c2

        lax.fori_loop(0, _K // 16, _scale, 0)

        pltpu.sync_copy(buf, acc.at[dst_v.at[j]], add=True)

    _gather(0, rows_a, sem_a).start()

    def _outer(i, carry):
        j = 2 * i
        _gather(j + 1, rows_b, sem_b).start()
        _process(j, rows_a, sem_a)
        _gather(j + 2, rows_a, sem_a).start()
        _process(j + 1, rows_b, sem_b)
        return carry

    lax.fori_loop(0, (_NCHUNK - 1) // 2, _outer, 0)
    _process(_NCHUNK - 1, rows_a, sem_a)
    plsc.subcore_barrier()

    pltpu.sync_copy(acc.at[pl.ds(s * _RPT, _RPT)],
                    out_hbm.at[c, pl.ds(s * _RPT, _RPT)])


_agg_kernel = functools.partial(
    pl.kernel,
    out_type=jax.ShapeDtypeStruct((_NC, _N, _D), jnp.float32),
    mesh=_mesh,
    compiler_params=_sc_params,
    scratch_types=[
        pltpu.VMEM((_NCHUNK, _K), jnp.int32),
        pltpu.VMEM((_NCHUNK, _K), jnp.int32),
        pltpu.VMEM((_EW,), jnp.float32),
        pltpu.VMEM((_K, _D), jnp.float32),
        pltpu.VMEM((_K, _D), jnp.float32),
        pltpu.SemaphoreType.DMA,
        pltpu.SemaphoreType.DMA,
        pltpu.VMEM_SHARED((_N, _D), jnp.float32),
    ],
)(_agg_body)


def _pre_body(degp_ref, x_ref, wfc_ref, w0_ref, dis_ref, y0_ref):
    ones = jnp.ones((_NW, 1), jnp.float32)
    deg = lax.dot_general(degp_ref[...], ones, (((0,), (0,)), ((), ())),
                          preferred_element_type=jnp.float32) + 1.0
    dis = jnp.where(deg > 0, lax.rsqrt(jnp.maximum(deg, 1e-12)), 0.0)
    h0 = jnp.dot(x_ref[...], wfc_ref[...].T, preferred_element_type=jnp.float32)
    y0 = dis * jnp.dot(h0, w0_ref[...].T, preferred_element_type=jnp.float32)
    dis_ref[...] = dis
    y0_ref[...] = y0


_pre_kernel = pl.pallas_call(
    _pre_body,
    out_shape=(
        jax.ShapeDtypeStruct((_N, 1), jnp.float32),
        jax.ShapeDtypeStruct((_N, _D), jnp.float32),
    ),
)


def _post_body(p_ref, y_ref, dis_ref, b_ref, g_ref, be_ref, wn_ref,
               h_ref, yn_ref):
    agg = p_ref[0] + p_ref[1] + y_ref[...]
    pre = dis_ref[...] * agg + b_ref[...]
    r = jnp.maximum(pre, 0.0)
    mu = jnp.mean(r, axis=0, keepdims=True)
    d = r - mu
    var = jnp.mean(d * d, axis=0, keepdims=True)
    h = g_ref[...] * d * lax.rsqrt(var + 1e-5) + be_ref[...]
    h_ref[...] = h
    yn_ref[...] = dis_ref[...] * jnp.dot(h, wn_ref[...].T,
                                         preferred_element_type=jnp.float32)


_post_kernel = pl.pallas_call(
    _post_body,
    out_shape=(
        jax.ShapeDtypeStruct((_N, _D), jnp.float32),
        jax.ShapeDtypeStruct((_N, _D), jnp.float32),
    ),
)


def _final_body(p_ref, y_ref, dis_ref, b_ref, g_ref, be_ref,
                batch_ref, h0_ref, h1_ref, h2_ref, xout_ref):
    agg = p_ref[0] + p_ref[1] + y_ref[...]
    pre = dis_ref[...] * agg + b_ref[...]
    r = jnp.maximum(pre, 0.0)
    mu = jnp.mean(r, axis=0, keepdims=True)
    d = r - mu
    var = jnp.mean(d * d, axis=0, keepdims=True)
    h2 = g_ref[...] * d * lax.rsqrt(var + 1e-5) + be_ref[...]
    h2_ref[...] = h2

    hsum = h0_ref[...] + h1_ref[...] + h2
    ids = lax.broadcasted_iota(jnp.int32, (_G, _N), 0)
    sel = (batch_ref[...] == ids).astype(jnp.float32)
    xout_ref[...] = jnp.dot(sel, hsum, preferred_element_type=jnp.float32)


_final_kernel = pl.pallas_call(
    _final_body,
    out_shape=(
        jax.ShapeDtypeStruct((_N, _D), jnp.float32),
        jax.ShapeDtypeStruct((_G, _D), jnp.float32),
    ),
)


@jax.jit
def kernel(x, edge_index, edge_weight, batch, W_fc,
           W0, b0, g0, be0, W1, b1, g1, be1, W2, b2, g2, be2):
    src3 = edge_index[0].reshape(_NW, _NCHUNK, _K)
    dst3 = edge_index[1].reshape(_NW, _NCHUNK, _K)

    degp = _deg_kernel(edge_index[1], edge_weight)
    dis, y0 = _pre_kernel(degp, x, W_fc, W0)

    p0 = _agg_kernel(y0, src3, dst3, edge_weight)
    h0, y1 = _post_kernel(p0, y0, dis, b0.reshape(1, _D), g0.reshape(1, _D),
                          be0.reshape(1, _D), W1)

    p1 = _agg_kernel(y1, src3, dst3, edge_weight)
    h1, y2 = _post_kernel(p1, y1, dis, b1.reshape(1, _D), g1.reshape(1, _D),
                          be1.reshape(1, _D), W2)

    p2 = _agg_kernel(y2, src3, dst3, edge_weight)
    h2, xout = _final_kernel(p2, y2, dis, b2.reshape(1, _D),
                             g2.reshape(1, _D), be2.reshape(1, _D),
                             batch.reshape(1, _N), h0, h1)

    return (xout, jnp.concatenate([h0, h1, h2], axis=1))
